# baseline (speedup 1.0000x reference)
import jax
import jax.numpy as jnp
from jax.experimental import pallas as pl

_N = 50
_H = 64
_SG = 100
_SN = 7
_GC = 4
_NEG = -1e9

_f32 = jnp.float32
_MXT = jnp.float32


def _dot(a, b):
    return jax.lax.dot_general(a, b, (((1,), (0,)), ((), ())),
                               preferred_element_type=_f32)


def _dott(a, b):
    return jax.lax.dot_general(a, b, (((1,), (1,)), ((), ())),
                               preferred_element_type=_f32)


def _iota(shape, dim):
    return jax.lax.broadcasted_iota(jnp.int32, shape, dim)


def _block_diag_l1(w, width, blocks):
    rows = width * blocks
    cols = _H * blocks
    rowsel = (_iota((rows, width), 1) == _iota((rows, width), 0) % width)
    t = _dott(rowsel.astype(_MXT), w.astype(_MXT))
    ht = (_iota((_H, cols), 0) == _iota((_H, cols), 1) % _H).astype(_MXT)
    full = _dot(t.astype(_MXT), ht)
    d = (_iota((rows, cols), 0) // width) == (_iota((rows, cols), 1) // _H)
    return jnp.where(d, full, 0.0).astype(_MXT)


def _block_diag_l2(w, blocks):
    n = _H * blocks
    rowsel = (_iota((n, _H), 1) == _iota((n, _H), 0) % _H)
    v = _dott(rowsel.astype(_MXT), w.astype(_MXT))
    ht = (_iota((_H, n), 0) == _iota((_H, n), 1) % _H).astype(_MXT)
    full = _dot(v.astype(_MXT), ht)
    d = (_iota((n, n), 0) // _H) == (_iota((n, n), 1) // _H)
    return jnp.where(d, full, 0.0).astype(_MXT)


def _tile_row(b, blocks):
    r = b.reshape(1, _H)
    return jnp.concatenate([r] * blocks, axis=1)


def _fold_blocks(acc, blocks):
    out = acc[:, 0:_H]
    for i in range(1, blocks):
        out = out + acc[:, i * _H:(i + 1) * _H]
    return out


def _body(x_ref, w11_ref, b11_ref, w21_ref, b21_ref, w12_ref, b12_ref,
          w22_ref, b22_ref, w13_ref, b13_ref, w23_ref, b23_ref,
          w3_ref, b3_ref, w4_ref, b4_ref, w5_ref, b5_ref, out_ref):
    relu = lambda v: jnp.maximum(v, 0.0)
    x = x_ref[...]

    xg = x[:, 32:232]
    zg = (xg == 0.0).astype(_f32)
    pz_g = (_iota((200, _SG), 0) // 2 == _iota((200, _SG), 1)).astype(_f32)
    zz_g = _dot(zg, pz_g)
    counts_g = jnp.sum(jnp.where(zz_g > 1.5, 0.0, 1.0), axis=1, keepdims=True)

    xn = x[:, 0:28]
    zn = (xn == 0.0).astype(_f32)
    pz_n = (_iota((28, _SN), 0) // 4 == _iota((28, _SN), 1)).astype(_f32)
    zz_n = _dot(zn, pz_n)
    counts_n = jnp.sum(jnp.where(zz_n > 3.5, 0.0, 1.0), axis=1, keepdims=True)

    lanes = _GC * _H
    bd1_g = _block_diag_l1(w13_ref[...], 2, _GC)
    bd2_g = _block_diag_l2(w23_ref[...], _GC)
    b1t_g = _tile_row(b13_ref[...], _GC)
    b2t_g = _tile_row(b23_ref[...], _GC)
    smap = (_iota((1, lanes), 1) // _H).astype(_f32)

    acc_g = jnp.zeros((_N, lanes), _f32)
    for k in range(_SG // _GC):
        xs = x[:, 32 + 8 * k:40 + 8 * k].astype(_MXT)
        h1 = relu(_dot(xs, bd1_g) + b1t_g)
        h2 = _dot(h1.astype(_MXT), bd2_g)
        mb = jnp.where(smap + float(_GC * k) < counts_g, b2t_g, _NEG)
        acc_g = acc_g + relu(h2 + mb)
    sum_grid = _fold_blocks(acc_g, _GC)

    bd1_n = _block_diag_l1(w11_ref[...], 4, _GC)
    bd2_n = _block_diag_l2(w21_ref[...], _GC)
    b1t_n = _tile_row(b11_ref[...], _GC)
    b2t_n = _tile_row(b21_ref[...], _GC)

    acc_n = jnp.zeros((_N, lanes), _f32)
    for k in range(2):
        base = 12 * k
        xs = x[:, base:base + 16].astype(_MXT)
        h1 = relu(_dot(xs, bd1_n) + b1t_n)
        h2 = _dot(h1.astype(_MXT), bd2_n)
        seg = smap + float(3 * k)
        ok = seg < counts_n
        if k == 1:
            ok = jnp.logical_and(ok, smap > 0.5)
        mb = jnp.where(ok, b2t_n, _NEG)
        acc_n = acc_n + relu(h2 + mb)
    sum_neigh = _fold_blocks(acc_n, _GC)

    xs = x[:, 28:32].astype(_MXT)
    h1 = relu(_dott(xs, w12_ref[...].astype(_MXT)) + b12_ref[...].reshape(1, _H))
    h_s = relu(_dott(h1.astype(_MXT), w22_ref[...].astype(_MXT))
               + b22_ref[...].reshape(1, _H))

    w3 = w3_ref[...].astype(_MXT)
    h3 = relu(_dott(sum_neigh.astype(_MXT), w3[:, 0:_H])
              + _dott(h_s.astype(_MXT), w3[:, _H:2 * _H])
              + _dott(sum_grid.astype(_MXT), w3[:, 2 * _H:3 * _H])
              + b3_ref[...].reshape(1, _H))
    h4 = relu(_dott(h3.astype(_MXT), w4_ref[...].astype(_MXT))
              + b4_ref[...].reshape(1, _H))
    out_ref[...] = (_dott(h4.astype(_MXT), w5_ref[...].astype(_MXT))
                    + b5_ref[...].reshape(1, 2))


def kernel(X, W1_1, b1_1, W2_1, b2_1, W1_2, b1_2, W2_2, b2_2,
           W1_3, b1_3, W2_3, b2_3, W3, b3, W4, b4, W5, b5):
    return pl.pallas_call(
        _body,
        out_shape=jax.ShapeDtypeStruct((_N, 2), _f32),
    )(X, W1_1, b1_1, W2_1, b2_1, W1_2, b1_2, W2_2, b2_2,
      W1_3, b1_3, W2_3, b2_3, W3, b3, W4, b4, W5, b5)
